# Initial kernel scaffold; baseline (speedup 1.0000x reference)
import jax
import jax.numpy as jnp
from jax.experimental import pallas as pl


def kernel(distribution, history):
    raise NotImplementedError("write your pallas kernel here")


# speedup vs baseline: 2.4579x; 2.4579x over previous
import jax
import jax.numpy as jnp
from jax.experimental import pallas as pl

N_USERS = 4096
N_ITEMS = 1000
H = 12
A = 1 << H
R = 256


def _body(dist_ref, hist_ref, o_ref, act_ref, num_ref):
    dist = dist_ref[...]
    hist = hist_ref[...]

    k_ids = jax.lax.broadcasted_iota(jnp.int32, (A, 128), 0)
    j_ids = jnp.minimum(jax.lax.broadcasted_iota(jnp.int32, (A, 128), 1), 31)
    bitmat = ((k_ids >> j_ids) & 1).astype(jnp.float32)
    eb = jnp.dot(dist, bitmat, preferred_element_type=jnp.float32)

    col_ids = jax.lax.broadcasted_iota(jnp.int32, (R, N_ITEMS), 1)
    o = jnp.zeros((R, N_ITEMS), jnp.float32)
    for j in range(H):
        o = o + jnp.where(hist[:, j : j + 1] == col_ids, eb[:, j : j + 1], 0.0)
    o_ref[...] = o

    act = o > 0.5
    act_ref[...] = act.astype(jnp.int8)

    pw = (1 << jax.lax.broadcasted_iota(jnp.int32, (R, H), 1)).astype(jnp.int32)
    num_ref[...] = jnp.sum(act[:, :H].astype(jnp.int32) * pw, axis=1, keepdims=True)


def kernel(distribution, history):
    hist = history.astype(jnp.int32)
    grid = (N_USERS // R,)
    o, act8, num = pl.pallas_call(
        _body,
        grid=grid,
        in_specs=[
            pl.BlockSpec((R, A), lambda i: (i, 0)),
            pl.BlockSpec((R, H), lambda i: (i, 0)),
        ],
        out_specs=[
            pl.BlockSpec((R, N_ITEMS), lambda i: (i, 0)),
            pl.BlockSpec((R, N_ITEMS), lambda i: (i, 0)),
            pl.BlockSpec((R, 1), lambda i: (i, 0)),
        ],
        out_shape=[
            jax.ShapeDtypeStruct((N_USERS, N_ITEMS), jnp.float32),
            jax.ShapeDtypeStruct((N_USERS, N_ITEMS), jnp.int8),
            jax.ShapeDtypeStruct((N_USERS, 1), jnp.int32),
        ],
    )(distribution, hist)
    return (o, act8.astype(jnp.bool_), num.reshape(N_USERS))
